# baseline (speedup 1.0000x reference)
import functools

import numpy as np
import jax
import jax.numpy as jnp
from jax import lax
from jax.experimental import pallas as pl
from jax.experimental.pallas import tpu as pltpu
from jax.experimental.pallas import tpu_sc as plsc

_N = 32768
_K = 2048
_L = 16
_NV = _N // _L
_ROWS = 64
_WORKERS = 32
_ROWS_PER_W = _ROWS // _WORKERS
_HSTRIDE = 2049
_CSTRIDE = 129

_MININT = np.int32(-0x80000000)


def _desc_key(x):
  b = plsc.bitcast(x, jnp.int32)
  neg = b < 0
  mono = jnp.where(neg, ~b, b | _MININT)
  return ~mono


def _key_to_val(kd):
  mono = ~kd
  b = jnp.where(mono < 0, mono ^ _MININT, ~mono)
  return plsc.bitcast(b, jnp.float32)


def _popcnt(mask):
  return plsc.all_reduce_population_count(mask)[0]


def _clear(ref, nwords, unroll=8):
  z = jnp.zeros((_L,), jnp.int32)
  nv = nwords // _L
  bulk = nv // unroll

  def body(i, _):
    for u in range(unroll):
      ref[pl.ds((i * unroll + u) * _L, _L)] = z
    return 0

  lax.fori_loop(0, bulk, body, 0)
  for v in range(bulk * unroll, nv):
    ref[pl.ds(v * _L, _L)] = z


def _scan_chunk(h, run, need):
  cum = run + plsc.cumsum(h)
  cross = cum >= need
  j = plsc.all_reduce_ffs(cross)[0]
  tot = cum[_L - 1]
  crossed = tot >= need
  below = jnp.maximum(jnp.max(jnp.where(cross, 0, cum)), run)
  return j, tot, crossed, below


def _find_bin2(h16_v, c16_v, nchunks, need):
  def coarse_sum(c2):
    t = c16_v[pl.ds(c2 * _L, _L)]
    for l in range(1, _L):
      t = t + c16_v[pl.ds(l * _CSTRIDE + c2 * _L, _L)]
    return t

  def body(c2, carry):
    found, cstar, cb, run = carry
    j, tot, crossed, below = _scan_chunk(coarse_sum(c2), run, need)
    newly = jnp.logical_and(crossed, jnp.logical_not(found))
    cstar = jnp.where(newly, c2 * _L + j, cstar)
    cb = jnp.where(newly, below, cb)
    found = jnp.logical_or(found, crossed)
    return found, cstar, cb, tot

  init = (jnp.bool_(False), jnp.int32(0), jnp.int32(0), jnp.int32(0))
  _, cstar, cb0, _ = lax.fori_loop(0, nchunks // _L, body, init)

  h = h16_v[pl.ds(cstar * _L, _L)]
  for l in range(1, _L):
    h = h + h16_v[pl.ds(l * _HSTRIDE + cstar * _L, _L)]
  j, _, _, below = _scan_chunk(h, cb0, need)
  return cstar * _L + j, below


def _find_bin(loader, nchunks, need):
  def body(c, carry):
    found, b, cb, run = carry
    j, tot, crossed, below = _scan_chunk(loader(c), run, need)
    newly = jnp.logical_and(crossed, jnp.logical_not(found))
    b = jnp.where(newly, c * _L + j, b)
    cb = jnp.where(newly, below, cb)
    found = jnp.logical_or(found, crossed)
    return found, b, cb, tot

  init = (jnp.bool_(False), jnp.int32(0), jnp.int32(0), jnp.int32(0))
  _, b, cb, _ = lax.fori_loop(0, nchunks, body, init)
  return b, cb


def _sc_topk_kernel(x_hbm, vals_hbm, inds_hbm,
                    row_v, cand_v, a_kd, a_idx, b_kd, b_idx,
                    h16_v, c16_v, hist_v, offs_v, vals_v):
  cid = lax.axis_index("c")
  sid = lax.axis_index("s")
  wid = sid * 2 + cid
  iota = lax.iota(jnp.int32, _L)
  lane_base = iota * _HSTRIDE
  clane_base = iota * _CSTRIDE
  ones = jnp.ones((_L,), jnp.int32)

  def hist_chunk(c):
    return hist_v[pl.ds(c * _L, _L)]

  for sub in range(_ROWS_PER_W):
    row = wid * _ROWS_PER_W + sub
    pltpu.sync_copy(x_hbm.at[row], row_v)

    _clear(h16_v, _L * _HSTRIDE)
    _clear(c16_v, _L * _CSTRIDE)

    def l1_body(i, _):
      for u in range(8):
        sl = pl.ds((8 * i + u) * _L, _L)
        kd = _desc_key(row_v[sl])
        row_v[sl] = plsc.bitcast(kd, jnp.float32)
        d1 = lax.shift_right_logical(kd, 21)
        plsc.addupdate_scatter(h16_v, [lane_base + d1], ones)
        plsc.addupdate_scatter(
            c16_v, [clane_base + lax.shift_right_logical(d1, 4)], ones)
      return 0

    lax.fori_loop(0, _NV // 8, l1_body, 0)
    b1, cb1 = _find_bin2(h16_v, c16_v, 2048 // _L, _K)
    need2 = _K - cb1

    def l2_body(i, carry):
      ptr_s, ptr_c = carry
      for u in range(2):
        v = 2 * i + u
        sl = pl.ds(v * _L, _L)
        kd = plsc.bitcast(row_v[sl], jnp.int32)
        d1 = lax.shift_right_logical(kd, 21)
        ms = d1 < b1
        mc = d1 == b1
        gidx = v * _L + iota
        plsc.store_compressed(a_kd.at[pl.ds(ptr_s, _L)], kd, mask=ms)
        plsc.store_compressed(a_idx.at[pl.ds(ptr_s, _L)], gidx, mask=ms)
        plsc.store_compressed(cand_v.at[pl.ds(ptr_c, _L)], gidx, mask=mc)
        ptr_s = ptr_s + _popcnt(ms)
        ptr_c = ptr_c + _popcnt(mc)
      return ptr_s, ptr_c

    ptr_s0, ptr_c = lax.fori_loop(0, _NV // 2, l2_body,
                                  (jnp.int32(0), jnp.int32(0)))
    cand_v[pl.ds(ptr_c, _L)] = jnp.zeros((_L,), jnp.int32)
    nv_c0 = lax.shift_right_logical(ptr_c + (_L - 1), 4)

    _clear(h16_v, _L * _HSTRIDE)
    _clear(c16_v, _L * _CSTRIDE)

    def l2h_body(i, _):
      sl = pl.ds(i * _L, _L)
      idx = cand_v[sl]
      kd = plsc.bitcast(plsc.load_gather(row_v, [idx]), jnp.int32)
      d2 = lax.shift_right_logical(kd, 10) & 0x7FF
      valid = (i * _L + iota) < ptr_c
      plsc.addupdate_scatter(h16_v, [lane_base + d2], ones, mask=valid)
      plsc.addupdate_scatter(
          c16_v, [clane_base + lax.shift_right_logical(d2, 4)], ones,
          mask=valid)
      return 0

    lax.fori_loop(0, nv_c0, l2h_body, 0)
    b2, cb2 = _find_bin2(h16_v, c16_v, 2048 // _L, need2)
    need3 = need2 - cb2
    nv_c = nv_c0

    _clear(hist_v, 1024)

    def l3_body(i, _):
      sl = pl.ds(i * _L, _L)
      idx = cand_v[sl]
      kd = plsc.bitcast(plsc.load_gather(row_v, [idx]), jnp.int32)
      d2 = lax.shift_right_logical(kd, 10) & 0x7FF
      d3 = kd & 0x3FF
      valid = (i * _L + iota) < ptr_c
      m = jnp.logical_and(valid, d2 == b2)
      cnt, last = plsc.scan_count(d3, m)
      plsc.addupdate_scatter(hist_v, [d3], cnt, mask=last)
      return 0

    lax.fori_loop(0, nv_c, l3_body, 0)
    b3, cb3 = _find_bin(hist_chunk, 1024 // _L, need3)

    t = lax.shift_left(b1, 21) | lax.shift_left(b2, 10) | b3
    r = cb1 + cb2 + cb3

    def c_body(i, carry):
      ptr_s, ptr_e = carry
      sl = pl.ds(i * _L, _L)
      idx = cand_v[sl]
      kd = plsc.bitcast(plsc.load_gather(row_v, [idx]), jnp.int32)
      d2 = lax.shift_right_logical(kd, 10) & 0x7FF
      d3 = kd & 0x3FF
      valid = (i * _L + iota) < ptr_c
      mst = jnp.logical_and(
          valid,
          jnp.logical_or(d2 < b2, jnp.logical_and(d2 == b2, d3 < b3)))
      meq = jnp.logical_and(valid, kd == t)
      full = ptr_e >= _K - r
      meq = jnp.logical_and(meq, jnp.logical_not(full))
      off_e = jnp.minimum(r + ptr_e, _K)
      plsc.store_compressed(a_kd.at[pl.ds(ptr_s, _L)], kd, mask=mst)
      plsc.store_compressed(a_idx.at[pl.ds(ptr_s, _L)], idx, mask=mst)
      plsc.store_compressed(a_kd.at[pl.ds(off_e, _L)], kd, mask=meq)
      plsc.store_compressed(a_idx.at[pl.ds(off_e, _L)], idx, mask=meq)
      return ptr_s + _popcnt(mst), ptr_e + _popcnt(meq)

    lax.fori_loop(0, nv_c, c_body, (ptr_s0, jnp.int32(0)))

    src = (a_kd, a_idx)
    dst = (b_kd, b_idx)
    for p in range(4):
      shift = 8 * p
      s_kd, s_idx = src
      d_kd, d_idx = dst

      def hclr_body(i, _):
        z = jnp.zeros((_L,), jnp.int32)
        for l in range(_L):
          h16_v[pl.ds(l * _HSTRIDE + i * _L, _L)] = z
        return 0

      lax.fori_loop(0, 256 // _L, hclr_body, 0)

      def h_body(i, _, s_kd=s_kd, shift=shift):
        for u in range(2):
          kd = s_kd[pl.ds((2 * i + u) * _L, _L)]
          d = lax.shift_right_logical(kd, shift) & 0xFF
          plsc.addupdate_scatter(h16_v, [lane_base + d], ones)
        return 0

      lax.fori_loop(0, _K // _L // 2, h_body, 0)

      def o_body(c, run, shift=shift):
        h = h16_v[pl.ds(c * _L, _L)]
        for l in range(1, _L):
          h = h + h16_v[pl.ds(l * _HSTRIDE + c * _L, _L)]
        cum = run + plsc.cumsum(h)
        offs_v[pl.ds(c * _L, _L)] = cum - h
        return cum[_L - 1]

      lax.fori_loop(0, 256 // _L, o_body, jnp.int32(0))

      def p_load(i, s_kd=s_kd, s_idx=s_idx, shift=shift):
        sl = pl.ds(i * _L, _L)
        kd = s_kd[sl]
        ix = s_idx[sl]
        d = lax.shift_right_logical(kd, shift) & 0xFF
        cnt, last = plsc.scan_count(d)
        return kd, ix, d, cnt, last

      def p_body(i, carry, d_kd=d_kd, d_idx=d_idx, p_load=p_load):
        kd, ix, d, cnt, last = carry
        nxt = p_load(jnp.minimum(i + 1, _K // _L - 1))
        offs = plsc.load_gather(offs_v, [d])
        dest = offs + cnt - 1
        plsc.store_scatter(d_kd, [dest], kd)
        plsc.store_scatter(d_idx, [dest], ix)
        plsc.addupdate_scatter(offs_v, [d], cnt, mask=last)
        return nxt

      lax.fori_loop(0, _K // _L, p_body, p_load(jnp.int32(0)))
      src, dst = dst, src

    def out_body(i, _):
      sl = pl.ds(i * _L, _L)
      vals_v[sl] = _key_to_val(a_kd[sl]) + jnp.float32(1.0)
      return 0

    lax.fori_loop(0, _K // _L, out_body, 0)
    pltpu.sync_copy(vals_v, vals_hbm.at[row])
    pltpu.sync_copy(a_idx.at[pl.ds(0, _K)], inds_hbm.at[row])


@functools.partial(
    pl.kernel,
    out_type=(
        jax.ShapeDtypeStruct((_ROWS, _K), jnp.float32),
        jax.ShapeDtypeStruct((_ROWS, _K), jnp.int32),
    ),
    mesh=plsc.VectorSubcoreMesh(core_axis_name="c", subcore_axis_name="s"),
    compiler_params=pltpu.CompilerParams(needs_layout_passes=False),
    scratch_types=[
        pltpu.VMEM((_N,), jnp.float32),
        pltpu.VMEM((_N + _L,), jnp.int32),
        pltpu.VMEM((_K + _L,), jnp.int32),
        pltpu.VMEM((_K + _L,), jnp.int32),
        pltpu.VMEM((_K,), jnp.int32),
        pltpu.VMEM((_K,), jnp.int32),
        pltpu.VMEM((_L * _HSTRIDE,), jnp.int32),
        pltpu.VMEM((_L * _CSTRIDE,), jnp.int32),
        pltpu.VMEM((2048,), jnp.int32),
        pltpu.VMEM((256,), jnp.int32),
        pltpu.VMEM((_K,), jnp.float32),
    ],
)
def _sc_topk(x_hbm, vals_hbm, inds_hbm, *scratch):
  _sc_topk_kernel(x_hbm, vals_hbm, inds_hbm, *scratch)


def kernel(x):
  vals, inds = _sc_topk(x)
  inds = inds.astype(jnp.int64) + jnp.ones((_ROWS, _K), dtype=jnp.int64)
  return vals, inds


# speedup vs baseline: 1.0326x; 1.0326x over previous
import functools

import numpy as np
import jax
import jax.numpy as jnp
from jax import lax
from jax.experimental import pallas as pl
from jax.experimental.pallas import tpu as pltpu
from jax.experimental.pallas import tpu_sc as plsc

_N = 32768
_K = 2048
_L = 16
_NV = _N // _L
_ROWS = 64
_WORKERS = 32
_ROWS_PER_W = _ROWS // _WORKERS
_HSTRIDE = 2049
_CSTRIDE = 129
_SB = 2064
_CB = 16400

_MININT = np.int32(-0x80000000)


def _desc_key(x):
  b = plsc.bitcast(x, jnp.int32)
  neg = b < 0
  mono = jnp.where(neg, ~b, b | _MININT)
  return ~mono


def _key_to_val(kd):
  mono = ~kd
  b = jnp.where(mono < 0, mono ^ _MININT, ~mono)
  return plsc.bitcast(b, jnp.float32)


def _popcnt(mask):
  return plsc.all_reduce_population_count(mask)[0]


def _clear(ref, nwords, unroll=8):
  z = jnp.zeros((_L,), jnp.int32)
  nv = nwords // _L
  bulk = nv // unroll

  def body(i, _):
    for u in range(unroll):
      ref[pl.ds((i * unroll + u) * _L, _L)] = z
    return 0

  lax.fori_loop(0, bulk, body, 0)
  for v in range(bulk * unroll, nv):
    ref[pl.ds(v * _L, _L)] = z


def _scan_chunk(h, run, need):
  cum = run + plsc.cumsum(h)
  cross = cum >= need
  j = plsc.all_reduce_ffs(cross)[0]
  tot = cum[_L - 1]
  crossed = tot >= need
  below = jnp.maximum(jnp.max(jnp.where(cross, 0, cum)), run)
  return j, tot, crossed, below


def _find_bin2(h16_v, c16_v, nchunks, need):
  def coarse_sum(c2):
    t = c16_v[pl.ds(c2 * _L, _L)]
    for l in range(1, _L):
      t = t + c16_v[pl.ds(l * _CSTRIDE + c2 * _L, _L)]
    return t

  def body(c2, carry):
    found, cstar, cb, run = carry
    j, tot, crossed, below = _scan_chunk(coarse_sum(c2), run, need)
    newly = jnp.logical_and(crossed, jnp.logical_not(found))
    cstar = jnp.where(newly, c2 * _L + j, cstar)
    cb = jnp.where(newly, below, cb)
    found = jnp.logical_or(found, crossed)
    return found, cstar, cb, tot

  init = (jnp.bool_(False), jnp.int32(0), jnp.int32(0), jnp.int32(0))
  _, cstar, cb0, _ = lax.fori_loop(0, nchunks // _L, body, init)

  h = h16_v[pl.ds(cstar * _L, _L)]
  for l in range(1, _L):
    h = h + h16_v[pl.ds(l * _HSTRIDE + cstar * _L, _L)]
  j, _, _, below = _scan_chunk(h, cb0, need)
  return cstar * _L + j, below


def _find_bin(loader, nchunks, need):
  def body(c, carry):
    found, b, cb, run = carry
    j, tot, crossed, below = _scan_chunk(loader(c), run, need)
    newly = jnp.logical_and(crossed, jnp.logical_not(found))
    b = jnp.where(newly, c * _L + j, b)
    cb = jnp.where(newly, below, cb)
    found = jnp.logical_or(found, crossed)
    return found, b, cb, tot

  init = (jnp.bool_(False), jnp.int32(0), jnp.int32(0), jnp.int32(0))
  _, b, cb, _ = lax.fori_loop(0, nchunks, body, init)
  return b, cb


def _sc_topk_kernel(x_hbm, vals_hbm, inds_hbm,
                    row_v, cand_v, a_kd, a_idx, b_kd, b_idx,
                    h16_v, c16_v, hist_v, offs_v, vals_v):
  cid = lax.axis_index("c")
  sid = lax.axis_index("s")
  wid = sid * 2 + cid
  iota = lax.iota(jnp.int32, _L)
  lane_base = iota * _HSTRIDE
  clane_base = iota * _CSTRIDE
  ones = jnp.ones((_L,), jnp.int32)

  def hist_chunk(c):
    return hist_v[pl.ds(c * _L, _L)]

  for sub in range(_ROWS_PER_W):
    row = wid * _ROWS_PER_W + sub
    pltpu.sync_copy(x_hbm.at[row], row_v)

    _clear(h16_v, _L * _HSTRIDE)
    _clear(c16_v, _L * _CSTRIDE)

    def l1_body(i, _):
      for u in range(4):
        sl = pl.ds((4 * i + u) * _L, _L)
        kd = _desc_key(row_v[sl])
        row_v[sl] = plsc.bitcast(kd, jnp.float32)
        d1 = lax.shift_right_logical(kd, 21)
        plsc.addupdate_scatter(h16_v, [lane_base + d1], ones)
        plsc.addupdate_scatter(
            c16_v, [clane_base + lax.shift_right_logical(d1, 4)], ones)
      return 0

    lax.fori_loop(0, _NV // 4, l1_body, 0)
    b1, cb1 = _find_bin2(h16_v, c16_v, 2048 // _L, _K)
    need2 = _K - cb1

    def l2_body(i, carry):
      ps1, pc1, ps2, pc2 = carry
      sl = pl.ds(i * _L, _L)
      kd = plsc.bitcast(row_v[sl], jnp.int32)
      d1 = lax.shift_right_logical(kd, 21)
      ms = d1 < b1
      mc = d1 == b1
      gidx = i * _L + iota
      plsc.store_compressed(a_kd.at[pl.ds(ps1, _L)], kd, mask=ms)
      plsc.store_compressed(a_idx.at[pl.ds(ps1, _L)], gidx, mask=ms)
      plsc.store_compressed(cand_v.at[pl.ds(pc1, _L)], gidx, mask=mc)
      v = _NV // 2 + i
      sl2 = pl.ds(v * _L, _L)
      kd2 = plsc.bitcast(row_v[sl2], jnp.int32)
      d1b = lax.shift_right_logical(kd2, 21)
      ms2 = d1b < b1
      mc2 = d1b == b1
      gidx2 = v * _L + iota
      plsc.store_compressed(a_kd.at[pl.ds(_SB + ps2, _L)], kd2, mask=ms2)
      plsc.store_compressed(a_idx.at[pl.ds(_SB + ps2, _L)], gidx2, mask=ms2)
      plsc.store_compressed(cand_v.at[pl.ds(_CB + pc2, _L)], gidx2, mask=mc2)
      return (ps1 + _popcnt(ms), pc1 + _popcnt(mc),
              ps2 + _popcnt(ms2), pc2 + _popcnt(mc2))

    z32 = jnp.int32(0)
    ps1, pc1, ps2, pc2 = lax.fori_loop(0, _NV // 2, l2_body,
                                       (z32, z32, z32, z32))

    def cp_s(j, _):
      a_kd[pl.ds(ps1 + j * _L, _L)] = a_kd[pl.ds(_SB + j * _L, _L)]
      a_idx[pl.ds(ps1 + j * _L, _L)] = a_idx[pl.ds(_SB + j * _L, _L)]
      return 0

    lax.fori_loop(0, lax.shift_right_logical(ps2 + (_L - 1), 4), cp_s, 0)

    def cp_c(j, _):
      cand_v[pl.ds(pc1 + j * _L, _L)] = cand_v[pl.ds(_CB + j * _L, _L)]
      return 0

    lax.fori_loop(0, lax.shift_right_logical(pc2 + (_L - 1), 4), cp_c, 0)

    ptr_s0 = ps1 + ps2
    ptr_c = pc1 + pc2
    cand_v[pl.ds(ptr_c, _L)] = jnp.zeros((_L,), jnp.int32)
    nv_c0 = lax.shift_right_logical(ptr_c + (_L - 1), 4)

    _clear(h16_v, _L * _HSTRIDE)
    _clear(c16_v, _L * _CSTRIDE)

    def l2h_body(i, _):
      sl = pl.ds(i * _L, _L)
      idx = cand_v[sl]
      kd = plsc.bitcast(plsc.load_gather(row_v, [idx]), jnp.int32)
      d2 = lax.shift_right_logical(kd, 10) & 0x7FF
      valid = (i * _L + iota) < ptr_c
      plsc.addupdate_scatter(h16_v, [lane_base + d2], ones, mask=valid)
      plsc.addupdate_scatter(
          c16_v, [clane_base + lax.shift_right_logical(d2, 4)], ones,
          mask=valid)
      return 0

    lax.fori_loop(0, nv_c0, l2h_body, 0)
    b2, cb2 = _find_bin2(h16_v, c16_v, 2048 // _L, need2)
    need3 = need2 - cb2
    nv_c = nv_c0

    _clear(hist_v, 1024)

    def l3_body(i, _):
      sl = pl.ds(i * _L, _L)
      idx = cand_v[sl]
      kd = plsc.bitcast(plsc.load_gather(row_v, [idx]), jnp.int32)
      d2 = lax.shift_right_logical(kd, 10) & 0x7FF
      d3 = kd & 0x3FF
      valid = (i * _L + iota) < ptr_c
      m = jnp.logical_and(valid, d2 == b2)
      cnt, last = plsc.scan_count(d3, m)
      plsc.addupdate_scatter(hist_v, [d3], cnt, mask=last)
      return 0

    lax.fori_loop(0, nv_c, l3_body, 0)
    b3, cb3 = _find_bin(hist_chunk, 1024 // _L, need3)

    t = lax.shift_left(b1, 21) | lax.shift_left(b2, 10) | b3
    r = cb1 + cb2 + cb3

    def c_body(i, carry):
      ptr_s, ptr_e = carry
      sl = pl.ds(i * _L, _L)
      idx = cand_v[sl]
      kd = plsc.bitcast(plsc.load_gather(row_v, [idx]), jnp.int32)
      d2 = lax.shift_right_logical(kd, 10) & 0x7FF
      d3 = kd & 0x3FF
      valid = (i * _L + iota) < ptr_c
      mst = jnp.logical_and(
          valid,
          jnp.logical_or(d2 < b2, jnp.logical_and(d2 == b2, d3 < b3)))
      meq = jnp.logical_and(valid, kd == t)
      full = ptr_e >= _K - r
      meq = jnp.logical_and(meq, jnp.logical_not(full))
      off_e = jnp.minimum(r + ptr_e, _K)
      plsc.store_compressed(a_kd.at[pl.ds(ptr_s, _L)], kd, mask=mst)
      plsc.store_compressed(a_idx.at[pl.ds(ptr_s, _L)], idx, mask=mst)
      plsc.store_compressed(a_kd.at[pl.ds(off_e, _L)], kd, mask=meq)
      plsc.store_compressed(a_idx.at[pl.ds(off_e, _L)], idx, mask=meq)
      return ptr_s + _popcnt(mst), ptr_e + _popcnt(meq)

    lax.fori_loop(0, nv_c, c_body, (ptr_s0, jnp.int32(0)))

    src = (a_kd, a_idx)
    dst = (b_kd, b_idx)
    for p in range(4):
      shift = 8 * p
      s_kd, s_idx = src
      d_kd, d_idx = dst

      def hclr_body(i, _):
        z = jnp.zeros((_L,), jnp.int32)
        for l in range(_L):
          h16_v[pl.ds(l * _HSTRIDE + i * _L, _L)] = z
        return 0

      lax.fori_loop(0, 256 // _L, hclr_body, 0)

      def h_body(i, _, s_kd=s_kd, shift=shift):
        for u in range(2):
          kd = s_kd[pl.ds((2 * i + u) * _L, _L)]
          d = lax.shift_right_logical(kd, shift) & 0xFF
          plsc.addupdate_scatter(h16_v, [lane_base + d], ones)
        return 0

      lax.fori_loop(0, _K // _L // 2, h_body, 0)

      def o_body(c, run, shift=shift):
        h = h16_v[pl.ds(c * _L, _L)]
        for l in range(1, _L):
          h = h + h16_v[pl.ds(l * _HSTRIDE + c * _L, _L)]
        cum = run + plsc.cumsum(h)
        offs_v[pl.ds(c * _L, _L)] = cum - h
        return cum[_L - 1]

      lax.fori_loop(0, 256 // _L, o_body, jnp.int32(0))

      def p_load(i, s_kd=s_kd, s_idx=s_idx, shift=shift):
        sl = pl.ds(i * _L, _L)
        kd = s_kd[sl]
        ix = s_idx[sl]
        d = lax.shift_right_logical(kd, shift) & 0xFF
        cnt, last = plsc.scan_count(d)
        return kd, ix, d, cnt, last

      def p_body(i, carry, d_kd=d_kd, d_idx=d_idx, p_load=p_load):
        kd, ix, d, cnt, last = carry
        nxt = p_load(jnp.minimum(i + 1, _K // _L - 1))
        offs = plsc.load_gather(offs_v, [d])
        dest = offs + cnt - 1
        plsc.store_scatter(d_kd, [dest], kd)
        plsc.store_scatter(d_idx, [dest], ix)
        plsc.addupdate_scatter(offs_v, [d], cnt, mask=last)
        return nxt

      lax.fori_loop(0, _K // _L, p_body, p_load(jnp.int32(0)))
      src, dst = dst, src

    def out_body(i, _):
      sl = pl.ds(i * _L, _L)
      vals_v[sl] = _key_to_val(a_kd[sl]) + jnp.float32(1.0)
      return 0

    lax.fori_loop(0, _K // _L, out_body, 0)
    pltpu.sync_copy(vals_v, vals_hbm.at[row])
    pltpu.sync_copy(a_idx.at[pl.ds(0, _K)], inds_hbm.at[row])


@functools.partial(
    pl.kernel,
    out_type=(
        jax.ShapeDtypeStruct((_ROWS, _K), jnp.float32),
        jax.ShapeDtypeStruct((_ROWS, _K), jnp.int32),
    ),
    mesh=plsc.VectorSubcoreMesh(core_axis_name="c", subcore_axis_name="s"),
    compiler_params=pltpu.CompilerParams(needs_layout_passes=False),
    scratch_types=[
        pltpu.VMEM((_N,), jnp.float32),
        pltpu.VMEM((2 * _CB + _L,), jnp.int32),
        pltpu.VMEM((2 * _SB,), jnp.int32),
        pltpu.VMEM((2 * _SB,), jnp.int32),
        pltpu.VMEM((_K,), jnp.int32),
        pltpu.VMEM((_K,), jnp.int32),
        pltpu.VMEM((_L * _HSTRIDE,), jnp.int32),
        pltpu.VMEM((_L * _CSTRIDE,), jnp.int32),
        pltpu.VMEM((2048,), jnp.int32),
        pltpu.VMEM((256,), jnp.int32),
        pltpu.VMEM((_K,), jnp.float32),
    ],
)
def _sc_topk(x_hbm, vals_hbm, inds_hbm, *scratch):
  _sc_topk_kernel(x_hbm, vals_hbm, inds_hbm, *scratch)


def kernel(x):
  vals, inds = _sc_topk(x)
  inds = inds.astype(jnp.int64) + jnp.ones((_ROWS, _K), dtype=jnp.int64)
  return vals, inds


# speedup vs baseline: 1.0403x; 1.0075x over previous
import functools

import numpy as np
import jax
import jax.numpy as jnp
from jax import lax
from jax.experimental import pallas as pl
from jax.experimental.pallas import tpu as pltpu
from jax.experimental.pallas import tpu_sc as plsc

_N = 32768
_K = 2048
_L = 16
_NV = _N // _L
_ROWS = 64
_WORKERS = 32
_ROWS_PER_W = _ROWS // _WORKERS
_HSTRIDE = 2049
_CSTRIDE = 129

_MININT = np.int32(-0x80000000)


def _desc_key(x):
  b = plsc.bitcast(x, jnp.int32)
  neg = b < 0
  mono = jnp.where(neg, ~b, b | _MININT)
  return ~mono


def _key_to_val(kd):
  mono = ~kd
  b = jnp.where(mono < 0, mono ^ _MININT, ~mono)
  return plsc.bitcast(b, jnp.float32)


def _popcnt(mask):
  return plsc.all_reduce_population_count(mask)[0]


def _clear(ref, nwords, unroll=8):
  z = jnp.zeros((_L,), jnp.int32)
  nv = nwords // _L
  bulk = nv // unroll

  def body(i, _):
    for u in range(unroll):
      ref[pl.ds((i * unroll + u) * _L, _L)] = z
    return 0

  lax.fori_loop(0, bulk, body, 0)
  for v in range(bulk * unroll, nv):
    ref[pl.ds(v * _L, _L)] = z


def _scan_chunk(h, run, need):
  cum = run + plsc.cumsum(h)
  cross = cum >= need
  j = plsc.all_reduce_ffs(cross)[0]
  tot = cum[_L - 1]
  crossed = tot >= need
  below = jnp.maximum(jnp.max(jnp.where(cross, 0, cum)), run)
  return j, tot, crossed, below


def _find_bin2(h16_v, c16_v, nchunks, need):
  def coarse_sum(c2):
    t = c16_v[pl.ds(c2 * _L, _L)]
    for l in range(1, _L):
      t = t + c16_v[pl.ds(l * _CSTRIDE + c2 * _L, _L)]
    return t

  def body(c2, carry):
    found, cstar, cb, run = carry
    j, tot, crossed, below = _scan_chunk(coarse_sum(c2), run, need)
    newly = jnp.logical_and(crossed, jnp.logical_not(found))
    cstar = jnp.where(newly, c2 * _L + j, cstar)
    cb = jnp.where(newly, below, cb)
    found = jnp.logical_or(found, crossed)
    return found, cstar, cb, tot

  init = (jnp.bool_(False), jnp.int32(0), jnp.int32(0), jnp.int32(0))
  _, cstar, cb0, _ = lax.fori_loop(0, nchunks // _L, body, init)

  h = h16_v[pl.ds(cstar * _L, _L)]
  for l in range(1, _L):
    h = h + h16_v[pl.ds(l * _HSTRIDE + cstar * _L, _L)]
  j, _, _, below = _scan_chunk(h, cb0, need)
  return cstar * _L + j, below


def _find_bin(loader, nchunks, need):
  def body(c, carry):
    found, b, cb, run = carry
    j, tot, crossed, below = _scan_chunk(loader(c), run, need)
    newly = jnp.logical_and(crossed, jnp.logical_not(found))
    b = jnp.where(newly, c * _L + j, b)
    cb = jnp.where(newly, below, cb)
    found = jnp.logical_or(found, crossed)
    return found, b, cb, tot

  init = (jnp.bool_(False), jnp.int32(0), jnp.int32(0), jnp.int32(0))
  _, b, cb, _ = lax.fori_loop(0, nchunks, body, init)
  return b, cb


def _sc_topk_kernel(x_hbm, vals_hbm, inds_hbm,
                    row_v, cand_v, a_kd, a_idx, b_kd, b_idx,
                    h16_v, c16_v, hist_v, offs_v, vals_v):
  cid = lax.axis_index("c")
  sid = lax.axis_index("s")
  wid = sid * 2 + cid
  iota = lax.iota(jnp.int32, _L)
  lane_base = iota * _HSTRIDE
  clane_base = iota * _CSTRIDE
  ones = jnp.ones((_L,), jnp.int32)

  def hist_chunk(c):
    return hist_v[pl.ds(c * _L, _L)]

  for sub in range(_ROWS_PER_W):
    row = wid * _ROWS_PER_W + sub
    pltpu.sync_copy(x_hbm.at[row], row_v)

    _clear(h16_v, _L * _HSTRIDE)

    def l1_body(i, _):
      for u in range(4):
        sl = pl.ds((4 * i + u) * _L, _L)
        kd = _desc_key(row_v[sl])
        row_v[sl] = plsc.bitcast(kd, jnp.float32)
        d1 = lax.shift_right_logical(kd, 21)
        plsc.addupdate_scatter(h16_v, [lane_base + d1], ones)
      return 0

    lax.fori_loop(0, _NV // 4, l1_body, 0)

    def h16_chunk(c):
      t = h16_v[pl.ds(c * _L, _L)]
      for l in range(1, _L):
        t = t + h16_v[pl.ds(l * _HSTRIDE + c * _L, _L)]
      return t

    b1, cb1 = _find_bin(h16_chunk, 2048 // _L, _K)
    need2 = _K - cb1

    def l2_body(i, carry):
      ptr_s, ptr_c = carry
      sl = pl.ds(i * _L, _L)
      kd = plsc.bitcast(row_v[sl], jnp.int32)
      d1 = lax.shift_right_logical(kd, 21)
      ms = d1 < b1
      mc = d1 == b1
      gidx = i * _L + iota
      plsc.store_compressed(a_kd.at[pl.ds(ptr_s, _L)], kd, mask=ms)
      plsc.store_compressed(a_idx.at[pl.ds(ptr_s, _L)], gidx, mask=ms)
      plsc.store_compressed(cand_v.at[pl.ds(ptr_c, _L)], gidx, mask=mc)
      return ptr_s + _popcnt(ms), ptr_c + _popcnt(mc)

    ptr_s0, ptr_c = lax.fori_loop(0, _NV, l2_body,
                                  (jnp.int32(0), jnp.int32(0)))
    cand_v[pl.ds(ptr_c, _L)] = jnp.zeros((_L,), jnp.int32)
    nv_c0 = lax.shift_right_logical(ptr_c + (_L - 1), 4)

    _clear(h16_v, _L * _HSTRIDE)
    _clear(c16_v, _L * _CSTRIDE)

    def l2h_body(i, _):
      sl = pl.ds(i * _L, _L)
      idx = cand_v[sl]
      kd = plsc.bitcast(plsc.load_gather(row_v, [idx]), jnp.int32)
      d2 = lax.shift_right_logical(kd, 10) & 0x7FF
      valid = (i * _L + iota) < ptr_c
      plsc.addupdate_scatter(h16_v, [lane_base + d2], ones, mask=valid)
      plsc.addupdate_scatter(
          c16_v, [clane_base + lax.shift_right_logical(d2, 4)], ones,
          mask=valid)
      return 0

    lax.fori_loop(0, nv_c0, l2h_body, 0)
    b2, cb2 = _find_bin2(h16_v, c16_v, 2048 // _L, need2)
    need3 = need2 - cb2
    nv_c = nv_c0

    _clear(hist_v, 1024)

    def l3_body(i, _):
      sl = pl.ds(i * _L, _L)
      idx = cand_v[sl]
      kd = plsc.bitcast(plsc.load_gather(row_v, [idx]), jnp.int32)
      d2 = lax.shift_right_logical(kd, 10) & 0x7FF
      d3 = kd & 0x3FF
      valid = (i * _L + iota) < ptr_c
      m = jnp.logical_and(valid, d2 == b2)
      cnt, last = plsc.scan_count(d3, m)
      plsc.addupdate_scatter(hist_v, [d3], cnt, mask=last)
      return 0

    lax.fori_loop(0, nv_c, l3_body, 0)
    b3, cb3 = _find_bin(hist_chunk, 1024 // _L, need3)

    t = lax.shift_left(b1, 21) | lax.shift_left(b2, 10) | b3
    r = cb1 + cb2 + cb3

    def c_body(i, carry):
      ptr_s, ptr_e = carry
      sl = pl.ds(i * _L, _L)
      idx = cand_v[sl]
      kd = plsc.bitcast(plsc.load_gather(row_v, [idx]), jnp.int32)
      d2 = lax.shift_right_logical(kd, 10) & 0x7FF
      d3 = kd & 0x3FF
      valid = (i * _L + iota) < ptr_c
      mst = jnp.logical_and(
          valid,
          jnp.logical_or(d2 < b2, jnp.logical_and(d2 == b2, d3 < b3)))
      meq = jnp.logical_and(valid, kd == t)
      full = ptr_e >= _K - r
      meq = jnp.logical_and(meq, jnp.logical_not(full))
      off_e = jnp.minimum(r + ptr_e, _K)
      plsc.store_compressed(a_kd.at[pl.ds(ptr_s, _L)], kd, mask=mst)
      plsc.store_compressed(a_idx.at[pl.ds(ptr_s, _L)], idx, mask=mst)
      plsc.store_compressed(a_kd.at[pl.ds(off_e, _L)], kd, mask=meq)
      plsc.store_compressed(a_idx.at[pl.ds(off_e, _L)], idx, mask=meq)
      return ptr_s + _popcnt(mst), ptr_e + _popcnt(meq)

    lax.fori_loop(0, nv_c, c_body, (ptr_s0, jnp.int32(0)))

    src = (a_kd, a_idx)
    dst = (b_kd, b_idx)
    for p in range(4):
      shift = 8 * p
      s_kd, s_idx = src
      d_kd, d_idx = dst

      def hclr_body(i, _):
        z = jnp.zeros((_L,), jnp.int32)
        for l in range(_L):
          h16_v[pl.ds(l * _HSTRIDE + i * _L, _L)] = z
        return 0

      lax.fori_loop(0, 256 // _L, hclr_body, 0)

      def h_body(i, _, s_kd=s_kd, shift=shift):
        for u in range(2):
          kd = s_kd[pl.ds((2 * i + u) * _L, _L)]
          d = lax.shift_right_logical(kd, shift) & 0xFF
          plsc.addupdate_scatter(h16_v, [lane_base + d], ones)
        return 0

      lax.fori_loop(0, _K // _L // 2, h_body, 0)

      def o_body(c, run, shift=shift):
        h = h16_v[pl.ds(c * _L, _L)]
        for l in range(1, _L):
          h = h + h16_v[pl.ds(l * _HSTRIDE + c * _L, _L)]
        cum = run + plsc.cumsum(h)
        offs_v[pl.ds(c * _L, _L)] = cum - h
        return cum[_L - 1]

      lax.fori_loop(0, 256 // _L, o_body, jnp.int32(0))

      def p_load(i, s_kd=s_kd, s_idx=s_idx, shift=shift):
        sl = pl.ds(i * _L, _L)
        kd = s_kd[sl]
        ix = s_idx[sl]
        d = lax.shift_right_logical(kd, shift) & 0xFF
        cnt, last = plsc.scan_count(d)
        return kd, ix, d, cnt, last

      def p_body(i, carry, d_kd=d_kd, d_idx=d_idx, p_load=p_load):
        kd, ix, d, cnt, last = carry
        nxt = p_load(jnp.minimum(i + 1, _K // _L - 1))
        offs = plsc.load_gather(offs_v, [d])
        dest = offs + cnt - 1
        plsc.store_scatter(d_kd, [dest], kd)
        plsc.store_scatter(d_idx, [dest], ix)
        plsc.addupdate_scatter(offs_v, [d], cnt, mask=last)
        return nxt

      lax.fori_loop(0, _K // _L, p_body, p_load(jnp.int32(0)))
      src, dst = dst, src

    def out_body(i, _):
      sl = pl.ds(i * _L, _L)
      vals_v[sl] = _key_to_val(a_kd[sl]) + jnp.float32(1.0)
      return 0

    lax.fori_loop(0, _K // _L, out_body, 0)
    pltpu.sync_copy(vals_v, vals_hbm.at[row])
    pltpu.sync_copy(a_idx.at[pl.ds(0, _K)], inds_hbm.at[row])


@functools.partial(
    pl.kernel,
    out_type=(
        jax.ShapeDtypeStruct((_ROWS, _K), jnp.float32),
        jax.ShapeDtypeStruct((_ROWS, _K), jnp.int32),
    ),
    mesh=plsc.VectorSubcoreMesh(core_axis_name="c", subcore_axis_name="s"),
    compiler_params=pltpu.CompilerParams(needs_layout_passes=False),
    scratch_types=[
        pltpu.VMEM((_N,), jnp.float32),
        pltpu.VMEM((_N + _L,), jnp.int32),
        pltpu.VMEM((_K + _L,), jnp.int32),
        pltpu.VMEM((_K + _L,), jnp.int32),
        pltpu.VMEM((_K,), jnp.int32),
        pltpu.VMEM((_K,), jnp.int32),
        pltpu.VMEM((_L * _HSTRIDE,), jnp.int32),
        pltpu.VMEM((_L * _CSTRIDE,), jnp.int32),
        pltpu.VMEM((2048,), jnp.int32),
        pltpu.VMEM((256,), jnp.int32),
        pltpu.VMEM((_K,), jnp.float32),
    ],
)
def _sc_topk(x_hbm, vals_hbm, inds_hbm, *scratch):
  _sc_topk_kernel(x_hbm, vals_hbm, inds_hbm, *scratch)


def kernel(x):
  vals, inds = _sc_topk(x)
  inds = inds.astype(jnp.int64) + jnp.ones((_ROWS, _K), dtype=jnp.int64)
  return vals, inds


# speedup vs baseline: 1.2708x; 1.2215x over previous
import functools

import numpy as np
import jax
import jax.numpy as jnp
from jax import lax
from jax.experimental import pallas as pl
from jax.experimental.pallas import tpu as pltpu
from jax.experimental.pallas import tpu_sc as plsc

_N = 32768
_K = 2048
_L = 16
_NV = _N // _L
_ROWS = 64
_WORKERS = 32
_ROWS_PER_W = _ROWS // _WORKERS
_HSTRIDE = 2049
_CSTRIDE = 129

_MININT = np.int32(-0x80000000)


def _desc_key(x):
  b = plsc.bitcast(x, jnp.int32)
  neg = b < 0
  mono = jnp.where(neg, ~b, b | _MININT)
  return ~mono


def _key_to_val(kd):
  mono = ~kd
  b = jnp.where(mono < 0, mono ^ _MININT, ~mono)
  return plsc.bitcast(b, jnp.float32)


def _popcnt(mask):
  return plsc.all_reduce_population_count(mask)[0]


def _clear(ref, nwords, unroll=8):
  z = jnp.zeros((_L,), jnp.int32)
  nv = nwords // _L
  bulk = nv // unroll

  def body(i, _):
    for u in range(unroll):
      ref[pl.ds((i * unroll + u) * _L, _L)] = z
    return 0

  lax.fori_loop(0, bulk, body, 0)
  for v in range(bulk * unroll, nv):
    ref[pl.ds(v * _L, _L)] = z


def _scan_chunk(h, run, need):
  cum = run + plsc.cumsum(h)
  cross = cum >= need
  j = plsc.all_reduce_ffs(cross)[0]
  tot = cum[_L - 1]
  crossed = tot >= need
  below = jnp.maximum(jnp.max(jnp.where(cross, 0, cum)), run)
  return j, tot, crossed, below


def _find_bin2(h16_v, c16_v, nchunks, need):
  def coarse_sum(c2):
    t = c16_v[pl.ds(c2 * _L, _L)]
    for l in range(1, _L):
      t = t + c16_v[pl.ds(l * _CSTRIDE + c2 * _L, _L)]
    return t

  def body(c2, carry):
    found, cstar, cb, run = carry
    j, tot, crossed, below = _scan_chunk(coarse_sum(c2), run, need)
    newly = jnp.logical_and(crossed, jnp.logical_not(found))
    cstar = jnp.where(newly, c2 * _L + j, cstar)
    cb = jnp.where(newly, below, cb)
    found = jnp.logical_or(found, crossed)
    return found, cstar, cb, tot

  init = (jnp.bool_(False), jnp.int32(0), jnp.int32(0), jnp.int32(0))
  _, cstar, cb0, _ = lax.fori_loop(0, nchunks // _L, body, init)

  h = h16_v[pl.ds(cstar * _L, _L)]
  for l in range(1, _L):
    h = h + h16_v[pl.ds(l * _HSTRIDE + cstar * _L, _L)]
  j, _, _, below = _scan_chunk(h, cb0, need)
  return cstar * _L + j, below


def _find_bin(loader, nchunks, need):
  def body(c, carry):
    found, b, cb, run = carry
    j, tot, crossed, below = _scan_chunk(loader(c), run, need)
    newly = jnp.logical_and(crossed, jnp.logical_not(found))
    b = jnp.where(newly, c * _L + j, b)
    cb = jnp.where(newly, below, cb)
    found = jnp.logical_or(found, crossed)
    return found, b, cb, tot

  init = (jnp.bool_(False), jnp.int32(0), jnp.int32(0), jnp.int32(0))
  _, b, cb, _ = lax.fori_loop(0, nchunks, body, init)
  return b, cb


def _sc_topk_kernel(x_hbm, vals_hbm, inds_hbm,
                    row_v, cand_v, a_kd, a_idx, b_kd, b_idx,
                    h16_v, c16_v, hist_v, offs_v, vals_v):
  cid = lax.axis_index("c")
  sid = lax.axis_index("s")
  wid = sid * 2 + cid
  iota = lax.iota(jnp.int32, _L)
  lane_base = iota * _HSTRIDE
  clane_base = iota * _CSTRIDE
  ones = jnp.ones((_L,), jnp.int32)

  def hist_chunk(c):
    return hist_v[pl.ds(c * _L, _L)]

  for sub in range(_ROWS_PER_W):
    row = wid * _ROWS_PER_W + sub
    pltpu.sync_copy(x_hbm.at[row], row_v)

    _clear(h16_v, _L * _HSTRIDE)

    def l1_load(i):
      return tuple(row_v[pl.ds((4 * i + u) * _L, _L)] for u in range(4))

    def l1_body(i, carry):
      xs = carry
      nxt = l1_load(jnp.minimum(i + 1, _NV // 4 - 1))
      for u in range(4):
        kd = _desc_key(xs[u])
        row_v[pl.ds((4 * i + u) * _L, _L)] = plsc.bitcast(kd, jnp.float32)
        d1 = lax.shift_right_logical(kd, 21)
        plsc.addupdate_scatter(h16_v, [lane_base + d1], ones)
      return nxt

    lax.fori_loop(0, _NV // 4, l1_body, l1_load(jnp.int32(0)))

    def h16_chunk(c):
      t = h16_v[pl.ds(c * _L, _L)]
      for l in range(1, _L):
        t = t + h16_v[pl.ds(l * _HSTRIDE + c * _L, _L)]
      return t

    b1, cb1 = _find_bin(h16_chunk, 2048 // _L, _K)
    need2 = _K - cb1

    def l2_body(i, carry):
      ptr_s, ptr_c = carry
      sl = pl.ds(i * _L, _L)
      kd = plsc.bitcast(row_v[sl], jnp.int32)
      d1 = lax.shift_right_logical(kd, 21)
      ms = d1 < b1
      mc = d1 == b1
      gidx = i * _L + iota
      plsc.store_compressed(a_kd.at[pl.ds(ptr_s, _L)], kd, mask=ms)
      plsc.store_compressed(a_idx.at[pl.ds(ptr_s, _L)], gidx, mask=ms)
      plsc.store_compressed(cand_v.at[pl.ds(ptr_c, _L)], gidx, mask=mc)
      return ptr_s + _popcnt(ms), ptr_c + _popcnt(mc)

    ptr_s0, ptr_c = lax.fori_loop(0, _NV, l2_body,
                                  (jnp.int32(0), jnp.int32(0)))
    cand_v[pl.ds(ptr_c, _L)] = jnp.zeros((_L,), jnp.int32)
    nv_c0 = lax.shift_right_logical(ptr_c + (_L - 1), 4)

    _clear(h16_v, _L * _HSTRIDE)
    _clear(c16_v, _L * _CSTRIDE)

    def l2h_body(i, _):
      sl = pl.ds(i * _L, _L)
      idx = cand_v[sl]
      kd = plsc.bitcast(plsc.load_gather(row_v, [idx]), jnp.int32)
      d2 = lax.shift_right_logical(kd, 10) & 0x7FF
      valid = (i * _L + iota) < ptr_c
      plsc.addupdate_scatter(h16_v, [lane_base + d2], ones, mask=valid)
      plsc.addupdate_scatter(
          c16_v, [clane_base + lax.shift_right_logical(d2, 4)], ones,
          mask=valid)
      return 0

    lax.fori_loop(0, nv_c0, l2h_body, 0)
    b2, cb2 = _find_bin2(h16_v, c16_v, 2048 // _L, need2)
    need3 = need2 - cb2
    nv_c = nv_c0

    _clear(hist_v, 1024)

    def l3_body(i, _):
      sl = pl.ds(i * _L, _L)
      idx = cand_v[sl]
      kd = plsc.bitcast(plsc.load_gather(row_v, [idx]), jnp.int32)
      d2 = lax.shift_right_logical(kd, 10) & 0x7FF
      d3 = kd & 0x3FF
      valid = (i * _L + iota) < ptr_c
      m = jnp.logical_and(valid, d2 == b2)
      cnt, last = plsc.scan_count(d3, m)
      plsc.addupdate_scatter(hist_v, [d3], cnt, mask=last)
      return 0

    lax.fori_loop(0, nv_c, l3_body, 0)
    b3, cb3 = _find_bin(hist_chunk, 1024 // _L, need3)

    t = lax.shift_left(b1, 21) | lax.shift_left(b2, 10) | b3
    r = cb1 + cb2 + cb3

    def c_body(i, carry):
      ptr_s, ptr_e = carry
      sl = pl.ds(i * _L, _L)
      idx = cand_v[sl]
      kd = plsc.bitcast(plsc.load_gather(row_v, [idx]), jnp.int32)
      d2 = lax.shift_right_logical(kd, 10) & 0x7FF
      d3 = kd & 0x3FF
      valid = (i * _L + iota) < ptr_c
      mst = jnp.logical_and(
          valid,
          jnp.logical_or(d2 < b2, jnp.logical_and(d2 == b2, d3 < b3)))
      meq = jnp.logical_and(valid, kd == t)
      full = ptr_e >= _K - r
      meq = jnp.logical_and(meq, jnp.logical_not(full))
      off_e = jnp.minimum(r + ptr_e, _K)
      plsc.store_compressed(a_kd.at[pl.ds(ptr_s, _L)], kd, mask=mst)
      plsc.store_compressed(a_idx.at[pl.ds(ptr_s, _L)], idx, mask=mst)
      plsc.store_compressed(a_kd.at[pl.ds(off_e, _L)], kd, mask=meq)
      plsc.store_compressed(a_idx.at[pl.ds(off_e, _L)], idx, mask=meq)
      return ptr_s + _popcnt(mst), ptr_e + _popcnt(meq)

    lax.fori_loop(0, nv_c, c_body, (ptr_s0, jnp.int32(0)))

    src = (a_kd, a_idx)
    dst = (b_kd, b_idx)
    for p in range(4):
      shift = 8 * p
      s_kd, s_idx = src
      d_kd, d_idx = dst

      def hclr_body(i, _):
        z = jnp.zeros((_L,), jnp.int32)
        for l in range(_L):
          h16_v[pl.ds(l * _HSTRIDE + i * _L, _L)] = z
        return 0

      lax.fori_loop(0, 256 // _L, hclr_body, 0)

      def h_body(i, _, s_kd=s_kd, shift=shift):
        for u in range(2):
          kd = s_kd[pl.ds((2 * i + u) * _L, _L)]
          d = lax.shift_right_logical(kd, shift) & 0xFF
          plsc.addupdate_scatter(h16_v, [lane_base + d], ones)
        return 0

      lax.fori_loop(0, _K // _L // 2, h_body, 0)

      def o_body(c, run, shift=shift):
        h = h16_v[pl.ds(c * _L, _L)]
        for l in range(1, _L):
          h = h + h16_v[pl.ds(l * _HSTRIDE + c * _L, _L)]
        cum = run + plsc.cumsum(h)
        offs_v[pl.ds(c * _L, _L)] = cum - h
        return cum[_L - 1]

      lax.fori_loop(0, 256 // _L, o_body, jnp.int32(0))

      def p_load(i, s_kd=s_kd, s_idx=s_idx, shift=shift):
        sl = pl.ds(i * _L, _L)
        kd = s_kd[sl]
        ix = s_idx[sl]
        d = lax.shift_right_logical(kd, shift) & 0xFF
        cnt, last = plsc.scan_count(d)
        return kd, ix, d, cnt, last

      def p_body(i, carry, d_kd=d_kd, d_idx=d_idx, p_load=p_load):
        kd, ix, d, cnt, last = carry
        nxt = p_load(jnp.minimum(i + 1, _K // _L - 1))
        offs = plsc.load_gather(offs_v, [d])
        dest = offs + cnt - 1
        plsc.store_scatter(d_kd, [dest], kd)
        plsc.store_scatter(d_idx, [dest], ix)
        plsc.addupdate_scatter(offs_v, [d], cnt, mask=last)
        return nxt

      lax.fori_loop(0, _K // _L, p_body, p_load(jnp.int32(0)))
      src, dst = dst, src

    def out_body(i, _):
      sl = pl.ds(i * _L, _L)
      vals_v[sl] = _key_to_val(a_kd[sl]) + jnp.float32(1.0)
      return 0

    lax.fori_loop(0, _K // _L, out_body, 0)
    pltpu.sync_copy(vals_v, vals_hbm.at[row])
    pltpu.sync_copy(a_idx.at[pl.ds(0, _K)], inds_hbm.at[row])


@functools.partial(
    pl.kernel,
    out_type=(
        jax.ShapeDtypeStruct((_ROWS, _K), jnp.float32),
        jax.ShapeDtypeStruct((_ROWS, _K), jnp.int32),
    ),
    mesh=plsc.VectorSubcoreMesh(core_axis_name="c", subcore_axis_name="s"),
    compiler_params=pltpu.CompilerParams(needs_layout_passes=False),
    scratch_types=[
        pltpu.VMEM((_N,), jnp.float32),
        pltpu.VMEM((_N + _L,), jnp.int32),
        pltpu.VMEM((_K + _L,), jnp.int32),
        pltpu.VMEM((_K + _L,), jnp.int32),
        pltpu.VMEM((_K,), jnp.int32),
        pltpu.VMEM((_K,), jnp.int32),
        pltpu.VMEM((_L * _HSTRIDE,), jnp.int32),
        pltpu.VMEM((_L * _CSTRIDE,), jnp.int32),
        pltpu.VMEM((2048,), jnp.int32),
        pltpu.VMEM((256,), jnp.int32),
        pltpu.VMEM((_K,), jnp.float32),
    ],
)
def _sc_topk(x_hbm, vals_hbm, inds_hbm, *scratch):
  _sc_topk_kernel(x_hbm, vals_hbm, inds_hbm, *scratch)


def kernel(x):
  vals, inds = _sc_topk(x)
  inds = inds.astype(jnp.int64) + jnp.ones((_ROWS, _K), dtype=jnp.int64)
  return vals, inds


# speedup vs baseline: 1.3067x; 1.0283x over previous
import functools

import numpy as np
import jax
import jax.numpy as jnp
from jax import lax
from jax.experimental import pallas as pl
from jax.experimental.pallas import tpu as pltpu
from jax.experimental.pallas import tpu_sc as plsc

_N = 32768
_K = 2048
_L = 16
_NV = _N // _L
_ROWS = 64
_WORKERS = 32
_ROWS_PER_W = _ROWS // _WORKERS
_HSTRIDE = 2049
_CSTRIDE = 129

_MININT = np.int32(-0x80000000)


def _desc_key(x):
  b = plsc.bitcast(x, jnp.int32)
  neg = b < 0
  mono = jnp.where(neg, ~b, b | _MININT)
  return ~mono


def _key_to_val(kd):
  mono = ~kd
  b = jnp.where(mono < 0, mono ^ _MININT, ~mono)
  return plsc.bitcast(b, jnp.float32)


def _popcnt(mask):
  return plsc.all_reduce_population_count(mask)[0]


def _clear(ref, nwords, unroll=8):
  z = jnp.zeros((_L,), jnp.int32)
  nv = nwords // _L
  bulk = nv // unroll

  def body(i, _):
    for u in range(unroll):
      ref[pl.ds((i * unroll + u) * _L, _L)] = z
    return 0

  lax.fori_loop(0, bulk, body, 0)
  for v in range(bulk * unroll, nv):
    ref[pl.ds(v * _L, _L)] = z


def _scan_chunk(h, run, need):
  cum = run + plsc.cumsum(h)
  cross = cum >= need
  j = plsc.all_reduce_ffs(cross)[0]
  tot = cum[_L - 1]
  crossed = tot >= need
  below = jnp.maximum(jnp.max(jnp.where(cross, 0, cum)), run)
  return j, tot, crossed, below


def _find_bin2(h16_v, c16_v, nchunks, need):
  def coarse_sum(c2):
    t = c16_v[pl.ds(c2 * _L, _L)]
    for l in range(1, _L):
      t = t + c16_v[pl.ds(l * _CSTRIDE + c2 * _L, _L)]
    return t

  def body(c2, carry):
    found, cstar, cb, run = carry
    j, tot, crossed, below = _scan_chunk(coarse_sum(c2), run, need)
    newly = jnp.logical_and(crossed, jnp.logical_not(found))
    cstar = jnp.where(newly, c2 * _L + j, cstar)
    cb = jnp.where(newly, below, cb)
    found = jnp.logical_or(found, crossed)
    return found, cstar, cb, tot

  init = (jnp.bool_(False), jnp.int32(0), jnp.int32(0), jnp.int32(0))
  _, cstar, cb0, _ = lax.fori_loop(0, nchunks // _L, body, init)

  h = h16_v[pl.ds(cstar * _L, _L)]
  for l in range(1, _L):
    h = h + h16_v[pl.ds(l * _HSTRIDE + cstar * _L, _L)]
  j, _, _, below = _scan_chunk(h, cb0, need)
  return cstar * _L + j, below


def _find_bin(loader, nchunks, need):
  def body(c, carry):
    found, b, cb, run = carry
    j, tot, crossed, below = _scan_chunk(loader(c), run, need)
    newly = jnp.logical_and(crossed, jnp.logical_not(found))
    b = jnp.where(newly, c * _L + j, b)
    cb = jnp.where(newly, below, cb)
    found = jnp.logical_or(found, crossed)
    return found, b, cb, tot

  init = (jnp.bool_(False), jnp.int32(0), jnp.int32(0), jnp.int32(0))
  _, b, cb, _ = lax.fori_loop(0, nchunks, body, init)
  return b, cb


def _sc_topk_kernel(x_hbm, vals_hbm, inds_hbm,
                    row_v, cand_v, a_kd, a_idx, b_kd, b_idx,
                    h16_v, c16_v, hist_v, offs_v, vals_v):
  cid = lax.axis_index("c")
  sid = lax.axis_index("s")
  wid = sid * 2 + cid
  iota = lax.iota(jnp.int32, _L)
  lane_base = iota * _HSTRIDE
  clane_base = iota * _CSTRIDE
  ones = jnp.ones((_L,), jnp.int32)

  def hist_chunk(c):
    return hist_v[pl.ds(c * _L, _L)]

  for sub in range(_ROWS_PER_W):
    row = wid * _ROWS_PER_W + sub
    pltpu.sync_copy(x_hbm.at[row], row_v)

    _clear(h16_v, _L * _HSTRIDE)

    def l1_load(i):
      return tuple(row_v[pl.ds((4 * i + u) * _L, _L)] for u in range(4))

    def l1_body(i, carry):
      xs = carry
      nxt = l1_load(jnp.minimum(i + 1, _NV // 4 - 1))
      for u in range(4):
        kd = _desc_key(xs[u])
        row_v[pl.ds((4 * i + u) * _L, _L)] = plsc.bitcast(kd, jnp.float32)
        d1 = lax.shift_right_logical(kd, 21)
        plsc.addupdate_scatter(h16_v, [lane_base + d1], ones)
      return nxt

    lax.fori_loop(0, _NV // 4, l1_body, l1_load(jnp.int32(0)))

    def h16_chunk(c):
      t = h16_v[pl.ds(c * _L, _L)]
      for l in range(1, _L):
        t = t + h16_v[pl.ds(l * _HSTRIDE + c * _L, _L)]
      return t

    b1, cb1 = _find_bin(h16_chunk, 2048 // _L, _K)
    need2 = _K - cb1

    def l2_load(i):
      kd = plsc.bitcast(row_v[pl.ds(i * _L, _L)], jnp.int32)
      d1 = lax.shift_right_logical(kd, 21)
      return kd, d1 < b1, d1 == b1

    def l2_body(i, carry):
      ptr_s, ptr_c, kd, ms, mc = carry
      nxt = l2_load(jnp.minimum(i + 1, _NV - 1))
      gidx = i * _L + iota
      plsc.store_compressed(a_kd.at[pl.ds(ptr_s, _L)], kd, mask=ms)
      plsc.store_compressed(a_idx.at[pl.ds(ptr_s, _L)], gidx, mask=ms)
      plsc.store_compressed(cand_v.at[pl.ds(ptr_c, _L)], gidx, mask=mc)
      return (ptr_s + _popcnt(ms), ptr_c + _popcnt(mc)) + nxt

    ptr_s0, ptr_c, _, _, _ = lax.fori_loop(
        0, _NV, l2_body, (jnp.int32(0), jnp.int32(0)) + l2_load(jnp.int32(0)))
    cand_v[pl.ds(ptr_c, _L)] = jnp.zeros((_L,), jnp.int32)
    nv_c0 = lax.shift_right_logical(ptr_c + (_L - 1), 4)

    _clear(h16_v, _L * _HSTRIDE)
    _clear(c16_v, _L * _CSTRIDE)

    def l2h_body(i, _):
      sl = pl.ds(i * _L, _L)
      idx = cand_v[sl]
      kd = plsc.bitcast(plsc.load_gather(row_v, [idx]), jnp.int32)
      d2 = lax.shift_right_logical(kd, 10) & 0x7FF
      valid = (i * _L + iota) < ptr_c
      plsc.addupdate_scatter(h16_v, [lane_base + d2], ones, mask=valid)
      plsc.addupdate_scatter(
          c16_v, [clane_base + lax.shift_right_logical(d2, 4)], ones,
          mask=valid)
      return 0

    lax.fori_loop(0, nv_c0, l2h_body, 0)
    b2, cb2 = _find_bin2(h16_v, c16_v, 2048 // _L, need2)
    need3 = need2 - cb2
    nv_c = nv_c0

    _clear(hist_v, 1024)

    def l3_body(i, _):
      sl = pl.ds(i * _L, _L)
      idx = cand_v[sl]
      kd = plsc.bitcast(plsc.load_gather(row_v, [idx]), jnp.int32)
      d2 = lax.shift_right_logical(kd, 10) & 0x7FF
      d3 = kd & 0x3FF
      valid = (i * _L + iota) < ptr_c
      m = jnp.logical_and(valid, d2 == b2)
      cnt, last = plsc.scan_count(d3, m)
      plsc.addupdate_scatter(hist_v, [d3], cnt, mask=last)
      return 0

    lax.fori_loop(0, nv_c, l3_body, 0)
    b3, cb3 = _find_bin(hist_chunk, 1024 // _L, need3)

    t = lax.shift_left(b1, 21) | lax.shift_left(b2, 10) | b3
    r = cb1 + cb2 + cb3

    def c_body(i, carry):
      ptr_s, ptr_e = carry
      sl = pl.ds(i * _L, _L)
      idx = cand_v[sl]
      kd = plsc.bitcast(plsc.load_gather(row_v, [idx]), jnp.int32)
      d2 = lax.shift_right_logical(kd, 10) & 0x7FF
      d3 = kd & 0x3FF
      valid = (i * _L + iota) < ptr_c
      mst = jnp.logical_and(
          valid,
          jnp.logical_or(d2 < b2, jnp.logical_and(d2 == b2, d3 < b3)))
      meq = jnp.logical_and(valid, kd == t)
      full = ptr_e >= _K - r
      meq = jnp.logical_and(meq, jnp.logical_not(full))
      off_e = jnp.minimum(r + ptr_e, _K)
      plsc.store_compressed(a_kd.at[pl.ds(ptr_s, _L)], kd, mask=mst)
      plsc.store_compressed(a_idx.at[pl.ds(ptr_s, _L)], idx, mask=mst)
      plsc.store_compressed(a_kd.at[pl.ds(off_e, _L)], kd, mask=meq)
      plsc.store_compressed(a_idx.at[pl.ds(off_e, _L)], idx, mask=meq)
      return ptr_s + _popcnt(mst), ptr_e + _popcnt(meq)

    lax.fori_loop(0, nv_c, c_body, (ptr_s0, jnp.int32(0)))

    src = (a_kd, a_idx)
    dst = (b_kd, b_idx)
    for p in range(4):
      shift = 8 * p
      s_kd, s_idx = src
      d_kd, d_idx = dst

      def hclr_body(i, _):
        z = jnp.zeros((_L,), jnp.int32)
        for l in range(_L):
          h16_v[pl.ds(l * _HSTRIDE + i * _L, _L)] = z
        return 0

      lax.fori_loop(0, 256 // _L, hclr_body, 0)

      def h_load(i, s_kd=s_kd):
        return tuple(s_kd[pl.ds((2 * i + u) * _L, _L)] for u in range(2))

      def h_body(i, carry, shift=shift, h_load=h_load):
        kds = carry
        nxt = h_load(jnp.minimum(i + 1, _K // _L // 2 - 1))
        for u in range(2):
          d = lax.shift_right_logical(kds[u], shift) & 0xFF
          plsc.addupdate_scatter(h16_v, [lane_base + d], ones)
        return nxt

      lax.fori_loop(0, _K // _L // 2, h_body, h_load(jnp.int32(0)))

      def o_body(c, run, shift=shift):
        h = h16_v[pl.ds(c * _L, _L)]
        for l in range(1, _L):
          h = h + h16_v[pl.ds(l * _HSTRIDE + c * _L, _L)]
        cum = run + plsc.cumsum(h)
        offs_v[pl.ds(c * _L, _L)] = cum - h
        return cum[_L - 1]

      lax.fori_loop(0, 256 // _L, o_body, jnp.int32(0))

      def p_load(i, s_kd=s_kd, s_idx=s_idx, shift=shift):
        sl = pl.ds(i * _L, _L)
        kd = s_kd[sl]
        ix = s_idx[sl]
        d = lax.shift_right_logical(kd, shift) & 0xFF
        cnt, last = plsc.scan_count(d)
        return kd, ix, d, cnt, last

      def p_body(i, carry, d_kd=d_kd, d_idx=d_idx, p_load=p_load):
        kd, ix, d, cnt, last = carry
        nxt = p_load(jnp.minimum(i + 1, _K // _L - 1))
        offs = plsc.load_gather(offs_v, [d])
        dest = offs + cnt - 1
        plsc.store_scatter(d_kd, [dest], kd)
        plsc.store_scatter(d_idx, [dest], ix)
        plsc.addupdate_scatter(offs_v, [d], cnt, mask=last)
        return nxt

      lax.fori_loop(0, _K // _L, p_body, p_load(jnp.int32(0)))
      src, dst = dst, src

    def out_body(i, _):
      sl = pl.ds(i * _L, _L)
      vals_v[sl] = _key_to_val(a_kd[sl]) + jnp.float32(1.0)
      return 0

    lax.fori_loop(0, _K // _L, out_body, 0)
    pltpu.sync_copy(vals_v, vals_hbm.at[row])
    pltpu.sync_copy(a_idx.at[pl.ds(0, _K)], inds_hbm.at[row])


@functools.partial(
    pl.kernel,
    out_type=(
        jax.ShapeDtypeStruct((_ROWS, _K), jnp.float32),
        jax.ShapeDtypeStruct((_ROWS, _K), jnp.int32),
    ),
    mesh=plsc.VectorSubcoreMesh(core_axis_name="c", subcore_axis_name="s"),
    compiler_params=pltpu.CompilerParams(needs_layout_passes=False),
    scratch_types=[
        pltpu.VMEM((_N,), jnp.float32),
        pltpu.VMEM((_N + _L,), jnp.int32),
        pltpu.VMEM((_K + _L,), jnp.int32),
        pltpu.VMEM((_K + _L,), jnp.int32),
        pltpu.VMEM((_K,), jnp.int32),
        pltpu.VMEM((_K,), jnp.int32),
        pltpu.VMEM((_L * _HSTRIDE,), jnp.int32),
        pltpu.VMEM((_L * _CSTRIDE,), jnp.int32),
        pltpu.VMEM((2048,), jnp.int32),
        pltpu.VMEM((256,), jnp.int32),
        pltpu.VMEM((_K,), jnp.float32),
    ],
)
def _sc_topk(x_hbm, vals_hbm, inds_hbm, *scratch):
  _sc_topk_kernel(x_hbm, vals_hbm, inds_hbm, *scratch)


def kernel(x):
  vals, inds = _sc_topk(x)
  inds = inds.astype(jnp.int64) + jnp.ones((_ROWS, _K), dtype=jnp.int64)
  return vals, inds


# speedup vs baseline: 1.3309x; 1.0186x over previous
import functools

import numpy as np
import jax
import jax.numpy as jnp
from jax import lax
from jax.experimental import pallas as pl
from jax.experimental.pallas import tpu as pltpu
from jax.experimental.pallas import tpu_sc as plsc

_N = 32768
_K = 2048
_L = 16
_NV = _N // _L
_ROWS = 64
_WORKERS = 32
_ROWS_PER_W = _ROWS // _WORKERS
_HSTRIDE = 2049
_CSTRIDE = 129

_MININT = np.int32(-0x80000000)


def _desc_key(x):
  b = plsc.bitcast(x, jnp.int32)
  neg = b < 0
  mono = jnp.where(neg, ~b, b | _MININT)
  return ~mono


def _key_to_val(kd):
  mono = ~kd
  b = jnp.where(mono < 0, mono ^ _MININT, ~mono)
  return plsc.bitcast(b, jnp.float32)


def _popcnt(mask):
  return plsc.all_reduce_population_count(mask)[0]


def _clear(ref, nwords, unroll=8):
  z = jnp.zeros((_L,), jnp.int32)
  nv = nwords // _L
  bulk = nv // unroll

  def body(i, _):
    for u in range(unroll):
      ref[pl.ds((i * unroll + u) * _L, _L)] = z
    return 0

  lax.fori_loop(0, bulk, body, 0)
  for v in range(bulk * unroll, nv):
    ref[pl.ds(v * _L, _L)] = z


def _scan_chunk(h, run, need):
  cum = run + plsc.cumsum(h)
  cross = cum >= need
  j = plsc.all_reduce_ffs(cross)[0]
  tot = cum[_L - 1]
  crossed = tot >= need
  below = jnp.maximum(jnp.max(jnp.where(cross, 0, cum)), run)
  return j, tot, crossed, below


def _find_bin2(h16_v, c16_v, nchunks, need):
  def coarse_sum(c2):
    t = c16_v[pl.ds(c2 * _L, _L)]
    for l in range(1, _L):
      t = t + c16_v[pl.ds(l * _CSTRIDE + c2 * _L, _L)]
    return t

  def body(c2, carry):
    found, cstar, cb, run = carry
    j, tot, crossed, below = _scan_chunk(coarse_sum(c2), run, need)
    newly = jnp.logical_and(crossed, jnp.logical_not(found))
    cstar = jnp.where(newly, c2 * _L + j, cstar)
    cb = jnp.where(newly, below, cb)
    found = jnp.logical_or(found, crossed)
    return found, cstar, cb, tot

  init = (jnp.bool_(False), jnp.int32(0), jnp.int32(0), jnp.int32(0))
  _, cstar, cb0, _ = lax.fori_loop(0, nchunks // _L, body, init)

  h = h16_v[pl.ds(cstar * _L, _L)]
  for l in range(1, _L):
    h = h + h16_v[pl.ds(l * _HSTRIDE + cstar * _L, _L)]
  j, _, _, below = _scan_chunk(h, cb0, need)
  return cstar * _L + j, below


def _find_bin(loader, nchunks, need):
  def body(c, carry):
    found, b, cb, run = carry
    j, tot, crossed, below = _scan_chunk(loader(c), run, need)
    newly = jnp.logical_and(crossed, jnp.logical_not(found))
    b = jnp.where(newly, c * _L + j, b)
    cb = jnp.where(newly, below, cb)
    found = jnp.logical_or(found, crossed)
    return found, b, cb, tot

  init = (jnp.bool_(False), jnp.int32(0), jnp.int32(0), jnp.int32(0))
  _, b, cb, _ = lax.fori_loop(0, nchunks, body, init)
  return b, cb


def _sc_topk_kernel(x_hbm, vals_hbm, inds_hbm,
                    row_v, cand_v, a_kd, a_idx, b_kd, b_idx,
                    h16_v, c16_v, hist_v, offs_v, vals_v):
  cid = lax.axis_index("c")
  sid = lax.axis_index("s")
  wid = sid * 2 + cid
  iota = lax.iota(jnp.int32, _L)
  lane_base = iota * _HSTRIDE
  clane_base = iota * _CSTRIDE
  ones = jnp.ones((_L,), jnp.int32)

  def hist_chunk(c):
    return hist_v[pl.ds(c * _L, _L)]

  for sub in range(_ROWS_PER_W):
    row = wid * _ROWS_PER_W + sub
    pltpu.sync_copy(x_hbm.at[row], row_v)

    _clear(h16_v, _L * _HSTRIDE)

    def l1_load(i):
      return tuple(row_v[pl.ds((4 * i + u) * _L, _L)] for u in range(4))

    def l1_body(i, carry):
      xs = carry
      nxt = l1_load(jnp.minimum(i + 1, _NV // 4 - 1))
      for u in range(4):
        kd = _desc_key(xs[u])
        row_v[pl.ds((4 * i + u) * _L, _L)] = plsc.bitcast(kd, jnp.float32)
        d1 = lax.shift_right_logical(kd, 21)
        plsc.addupdate_scatter(h16_v, [lane_base + d1], ones)
      return nxt

    lax.fori_loop(0, _NV // 4, l1_body, l1_load(jnp.int32(0)))

    def h16_chunk(c):
      t = h16_v[pl.ds(c * _L, _L)]
      for l in range(1, _L):
        t = t + h16_v[pl.ds(l * _HSTRIDE + c * _L, _L)]
      return t

    b1, cb1 = _find_bin(h16_chunk, 2048 // _L, _K)
    need2 = _K - cb1

    def l2_load(i):
      kd = plsc.bitcast(row_v[pl.ds(i * _L, _L)], jnp.int32)
      d1 = lax.shift_right_logical(kd, 21)
      return kd, d1 < b1, d1 == b1

    def l2_body(i, carry):
      ptr_s, ptr_c, kd, ms, mc = carry
      nxt = l2_load(jnp.minimum(i + 1, _NV - 1))
      gidx = i * _L + iota
      plsc.store_compressed(a_kd.at[pl.ds(ptr_s, _L)], kd, mask=ms)
      plsc.store_compressed(a_idx.at[pl.ds(ptr_s, _L)], gidx, mask=ms)
      plsc.store_compressed(cand_v.at[pl.ds(ptr_c, _L)], gidx, mask=mc)
      return (ptr_s + _popcnt(ms), ptr_c + _popcnt(mc)) + nxt

    ptr_s0, ptr_c, _, _, _ = lax.fori_loop(
        0, _NV, l2_body, (jnp.int32(0), jnp.int32(0)) + l2_load(jnp.int32(0)))
    cand_v[pl.ds(ptr_c, _L)] = jnp.zeros((_L,), jnp.int32)
    nv_c0 = lax.shift_right_logical(ptr_c + (_L - 1), 4)

    _clear(h16_v, _L * _HSTRIDE)
    _clear(c16_v, _L * _CSTRIDE)

    def cand_load(i):
      idx = cand_v[pl.ds(i * _L, _L)]
      kd = plsc.bitcast(plsc.load_gather(row_v, [idx]), jnp.int32)
      return idx, kd, (i * _L + iota) < ptr_c

    def l2h_body(i, carry):
      _, kd, valid = carry
      nxt = cand_load(jnp.minimum(i + 1, nv_c0 - 1))
      d2 = lax.shift_right_logical(kd, 10) & 0x7FF
      plsc.addupdate_scatter(h16_v, [lane_base + d2], ones, mask=valid)
      plsc.addupdate_scatter(
          c16_v, [clane_base + lax.shift_right_logical(d2, 4)], ones,
          mask=valid)
      return nxt

    lax.fori_loop(0, nv_c0, l2h_body, cand_load(jnp.int32(0)))
    b2, cb2 = _find_bin2(h16_v, c16_v, 2048 // _L, need2)
    need3 = need2 - cb2
    nv_c = nv_c0

    _clear(hist_v, 1024)

    def l3_load(i):
      idx, kd, valid = cand_load(i)
      d2 = lax.shift_right_logical(kd, 10) & 0x7FF
      d3 = kd & 0x3FF
      m = jnp.logical_and(valid, d2 == b2)
      cnt, last = plsc.scan_count(d3, m)
      return d3, cnt, last

    def l3_body(i, carry):
      d3, cnt, last = carry
      nxt = l3_load(jnp.minimum(i + 1, nv_c - 1))
      plsc.addupdate_scatter(hist_v, [d3], cnt, mask=last)
      return nxt

    lax.fori_loop(0, nv_c, l3_body, l3_load(jnp.int32(0)))
    b3, cb3 = _find_bin(hist_chunk, 1024 // _L, need3)

    t = lax.shift_left(b1, 21) | lax.shift_left(b2, 10) | b3
    r = cb1 + cb2 + cb3

    def c_load(i):
      idx, kd, valid = cand_load(i)
      d2 = lax.shift_right_logical(kd, 10) & 0x7FF
      d3 = kd & 0x3FF
      mst = jnp.logical_and(
          valid,
          jnp.logical_or(d2 < b2, jnp.logical_and(d2 == b2, d3 < b3)))
      meq = jnp.logical_and(valid, kd == t)
      return idx, kd, mst, meq

    def c_body(i, carry):
      ptr_s, ptr_e, idx, kd, mst, meq = carry
      nxt = c_load(jnp.minimum(i + 1, nv_c - 1))
      full = ptr_e >= _K - r
      meq = jnp.logical_and(meq, jnp.logical_not(full))
      off_e = jnp.minimum(r + ptr_e, _K)
      plsc.store_compressed(a_kd.at[pl.ds(ptr_s, _L)], kd, mask=mst)
      plsc.store_compressed(a_idx.at[pl.ds(ptr_s, _L)], idx, mask=mst)
      plsc.store_compressed(a_kd.at[pl.ds(off_e, _L)], kd, mask=meq)
      plsc.store_compressed(a_idx.at[pl.ds(off_e, _L)], idx, mask=meq)
      return (ptr_s + _popcnt(mst), ptr_e + _popcnt(meq)) + nxt

    lax.fori_loop(0, nv_c, c_body,
                  (ptr_s0, jnp.int32(0)) + c_load(jnp.int32(0)))

    src = (a_kd, a_idx)
    dst = (b_kd, b_idx)
    for p in range(4):
      shift = 8 * p
      s_kd, s_idx = src
      d_kd, d_idx = dst

      def hclr_body(i, _):
        z = jnp.zeros((_L,), jnp.int32)
        for l in range(_L):
          h16_v[pl.ds(l * _HSTRIDE + i * _L, _L)] = z
        return 0

      lax.fori_loop(0, 256 // _L, hclr_body, 0)

      def h_load(i, s_kd=s_kd):
        return tuple(s_kd[pl.ds((2 * i + u) * _L, _L)] for u in range(2))

      def h_body(i, carry, shift=shift, h_load=h_load):
        kds = carry
        nxt = h_load(jnp.minimum(i + 1, _K // _L // 2 - 1))
        for u in range(2):
          d = lax.shift_right_logical(kds[u], shift) & 0xFF
          plsc.addupdate_scatter(h16_v, [lane_base + d], ones)
        return nxt

      lax.fori_loop(0, _K // _L // 2, h_body, h_load(jnp.int32(0)))

      def o_body(c, run, shift=shift):
        h = h16_v[pl.ds(c * _L, _L)]
        for l in range(1, _L):
          h = h + h16_v[pl.ds(l * _HSTRIDE + c * _L, _L)]
        cum = run + plsc.cumsum(h)
        offs_v[pl.ds(c * _L, _L)] = cum - h
        return cum[_L - 1]

      lax.fori_loop(0, 256 // _L, o_body, jnp.int32(0))

      def p_load(i, s_kd=s_kd, s_idx=s_idx, shift=shift):
        sl = pl.ds(i * _L, _L)
        kd = s_kd[sl]
        ix = s_idx[sl]
        d = lax.shift_right_logical(kd, shift) & 0xFF
        cnt, last = plsc.scan_count(d)
        return kd, ix, d, cnt, last

      def p_body(i, carry, d_kd=d_kd, d_idx=d_idx, p_load=p_load):
        kd, ix, d, cnt, last = carry
        nxt = p_load(jnp.minimum(i + 1, _K // _L - 1))
        offs = plsc.load_gather(offs_v, [d])
        dest = offs + cnt - 1
        plsc.store_scatter(d_kd, [dest], kd)
        plsc.store_scatter(d_idx, [dest], ix)
        plsc.addupdate_scatter(offs_v, [d], cnt, mask=last)
        return nxt

      lax.fori_loop(0, _K // _L, p_body, p_load(jnp.int32(0)))
      src, dst = dst, src

    def out_body(i, _):
      sl = pl.ds(i * _L, _L)
      vals_v[sl] = _key_to_val(a_kd[sl]) + jnp.float32(1.0)
      return 0

    lax.fori_loop(0, _K // _L, out_body, 0)
    pltpu.sync_copy(vals_v, vals_hbm.at[row])
    pltpu.sync_copy(a_idx.at[pl.ds(0, _K)], inds_hbm.at[row])


@functools.partial(
    pl.kernel,
    out_type=(
        jax.ShapeDtypeStruct((_ROWS, _K), jnp.float32),
        jax.ShapeDtypeStruct((_ROWS, _K), jnp.int32),
    ),
    mesh=plsc.VectorSubcoreMesh(core_axis_name="c", subcore_axis_name="s"),
    compiler_params=pltpu.CompilerParams(needs_layout_passes=False),
    scratch_types=[
        pltpu.VMEM((_N,), jnp.float32),
        pltpu.VMEM((_N + _L,), jnp.int32),
        pltpu.VMEM((_K + _L,), jnp.int32),
        pltpu.VMEM((_K + _L,), jnp.int32),
        pltpu.VMEM((_K,), jnp.int32),
        pltpu.VMEM((_K,), jnp.int32),
        pltpu.VMEM((_L * _HSTRIDE,), jnp.int32),
        pltpu.VMEM((_L * _CSTRIDE,), jnp.int32),
        pltpu.VMEM((2048,), jnp.int32),
        pltpu.VMEM((256,), jnp.int32),
        pltpu.VMEM((_K,), jnp.float32),
    ],
)
def _sc_topk(x_hbm, vals_hbm, inds_hbm, *scratch):
  _sc_topk_kernel(x_hbm, vals_hbm, inds_hbm, *scratch)


def kernel(x):
  vals, inds = _sc_topk(x)
  inds = inds.astype(jnp.int64) + jnp.ones((_ROWS, _K), dtype=jnp.int64)
  return vals, inds
